# Initial kernel scaffold; baseline (speedup 1.0000x reference)
import jax
import jax.numpy as jnp
from jax.experimental import pallas as pl


def kernel(x, centroids):
    raise NotImplementedError("write your pallas kernel here")


# speedup vs baseline: 1.3416x; 1.3416x over previous
import functools

import jax
import jax.numpy as jnp
from jax.experimental import pallas as pl
from jax.experimental.pallas import tpu as pltpu

_NUM_CLUSTERS = 8
_EPS = 1e-05
_N = 16384
_D = 1024
_BLK = 512


def _acts(xn):
    sig = jax.nn.sigmoid(xn)
    elu = jnp.where(xn > 0, xn, jnp.exp(jnp.minimum(xn, 0.0)) - 1.0)
    sp = jnp.maximum(xn, 0.0) + jnp.log(1.0 + jnp.exp(-jnp.abs(xn)))
    return [
        jax.nn.relu(xn),
        jax.nn.gelu(xn),
        jnp.tanh(xn),
        xn * sig,
        sig,
        jnp.clip(xn, 0.0, 6.0),
        elu,
        sp,
    ]


def _body(x_ref, c_ref, o_ref):
    xb = x_ref[...]
    c = c_ref[...]
    dots = jax.lax.dot_general(
        xb, c, (((1,), (1,)), ((), ())), preferred_element_type=jnp.float32
    )
    c2 = jnp.sum(c * c, axis=1)
    dist = c2[None, :] - 2.0 * dots
    labels = jnp.argmin(dist, axis=1)

    mean = jnp.mean(xb, axis=1, keepdims=True)
    xc = xb - mean
    var = jnp.sum(xc * xc, axis=1, keepdims=True) * (1.0 / (_D - 1))
    xn = xc * jax.lax.rsqrt(var + _EPS)

    out = jnp.zeros_like(xn)
    acts = _acts(xn)
    for k in range(_NUM_CLUSTERS):
        out = jnp.where((labels == k)[:, None], acts[k], out)
    o_ref[...] = out


@jax.jit
def kernel(x, centroids):
    return pl.pallas_call(
        _body,
        grid=(_N // _BLK,),
        in_specs=[
            pl.BlockSpec((_BLK, _D), lambda i: (i, 0)),
            pl.BlockSpec((_NUM_CLUSTERS, _D), lambda i: (0, 0)),
        ],
        out_specs=pl.BlockSpec((_BLK, _D), lambda i: (i, 0)),
        out_shape=jax.ShapeDtypeStruct((_N, _D), jnp.float32),
    )(x, centroids)
